# baseline (speedup 1.0000x reference)
import functools

import jax
import jax.numpy as jnp
from jax import lax
from jax.experimental import pallas as pl
from jax.experimental.pallas import tpu as pltpu
from jax.experimental.pallas import tpu_sc as plsc

_B = 4 * 4096
_D = 1024
_NC = 2
_NS = 16
_NW = _NC * _NS
_BPW = _B // _NW
_C = 16
_NCHUNK = _BPW // _C
_NBUF = 4
_NITER = _NCHUNK // _NBUF


def _emb_body(idx_hbm, table_hbm, out_hbm, idx_v, *rest):
    bufs = rest[:_NBUF]
    gsems = rest[_NBUF:2 * _NBUF]
    osems = rest[2 * _NBUF:3 * _NBUF]
    wid = lax.axis_index("s") * _NC + lax.axis_index("c")
    base = wid * _BPW
    pltpu.sync_copy(idx_hbm.at[wid // 8, pl.ds((wid % 8) * _BPW, _BPW)], idx_v)

    def gather(c, k):
        pltpu.async_copy(
            table_hbm.at[idx_v.at[pl.ds(pl.multiple_of(c * _C, 8), _C)]],
            bufs[k], gsems[k])

    def gather_wait(k):
        pltpu.make_async_copy(table_hbm.at[pl.ds(0, _C)], bufs[k],
                              gsems[k]).wait()

    def writeout(c, k):
        pltpu.async_copy(
            bufs[k], out_hbm.at[pl.ds(pl.multiple_of(base + c * _C, 8), _C)],
            osems[k])

    def writeout_wait(k):
        pltpu.make_async_copy(bufs[k], out_hbm.at[pl.ds(0, _C)],
                              osems[k]).wait()

    for k in range(_NBUF):
        gather(k, k)

    def loop_body(i, carry):
        for k in range(_NBUF):
            gather_wait(k)
            writeout((i - 1) * _NBUF + k, k)
        for k in range(_NBUF):
            writeout_wait(k)
            gather(i * _NBUF + k, k)
        return carry

    lax.fori_loop(1, _NITER, loop_body, 0)

    for k in range(_NBUF):
        gather_wait(k)
        writeout((_NITER - 1) * _NBUF + k, k)
    for k in range(_NBUF):
        writeout_wait(k)


@functools.partial(jax.jit, static_argnames=())
def kernel(input_ids, word_embeddings):
    mesh = plsc.VectorSubcoreMesh(core_axis_name="c", subcore_axis_name="s")
    run = pl.kernel(
        _emb_body,
        out_type=jax.ShapeDtypeStruct((_B, _D), jnp.float32),
        mesh=mesh,
        scratch_types=(
            [pltpu.VMEM((_BPW,), jnp.int32)]
            + [pltpu.VMEM((_C, _D), jnp.float32)] * _NBUF
            + [pltpu.SemaphoreType.DMA] * (2 * _NBUF)
        ),
    )
    out = run(input_ids, word_embeddings)
    return out.reshape(input_ids.shape + (_D,))


# speedup vs baseline: 1.0519x; 1.0519x over previous
import functools

import jax
import jax.numpy as jnp
from jax import lax
from jax.experimental import pallas as pl
from jax.experimental.pallas import tpu as pltpu
from jax.experimental.pallas import tpu_sc as plsc

_B = 4 * 4096
_D = 1024
_NC = 2
_NS = 16
_NW = _NC * _NS
_BPW = _B // _NW
_C = 16
_NCHUNK = _BPW // _C
_NBUF = 7
_GDEPTH = 6


def _emb_body(idx_hbm, table_hbm, out_hbm, idx_v, *rest):
    bufs = rest[:_NBUF]
    gsems = rest[_NBUF:2 * _NBUF]
    osems = rest[2 * _NBUF:3 * _NBUF]
    wid = lax.axis_index("s") * _NC + lax.axis_index("c")
    base = wid * _BPW
    irow, icol = wid // 8, (wid % 8) * _BPW
    pltpu.sync_copy(idx_hbm.at[irow, pl.ds(icol, 128)], idx_v.at[pl.ds(0, 128)])
    rest_cp = pltpu.async_copy(
        idx_hbm.at[irow, pl.ds(icol + 128, _BPW - 128)],
        idx_v.at[pl.ds(128, _BPW - 128)], osems[0])

    ghandles = [None] * _NCHUNK
    ohandles = [None] * _NCHUNK

    def writeout(g):
        ghandles[g].wait()
        flat = base + g * _C
        ohandles[g] = pltpu.async_copy(
            bufs[g % _NBUF],
            out_hbm.at[flat // 4096, pl.ds(flat % 4096, _C)],
            osems[g % _NBUF])

    for c in range(_NCHUNK):
        if c == 128 // _C:
            rest_cp.wait()
        if c >= _NBUF:
            ohandles[c - _NBUF].wait()
        ghandles[c] = pltpu.async_copy(
            table_hbm.at[idx_v.at[pl.ds(c * _C, _C)]], bufs[c % _NBUF],
            gsems[c % _NBUF])
        if c >= _GDEPTH - 1:
            writeout(c - (_GDEPTH - 1))
    for g in range(_NCHUNK - (_GDEPTH - 1), _NCHUNK):
        writeout(g)
    for g in range(_NCHUNK - _NBUF, _NCHUNK):
        ohandles[g].wait()


@functools.partial(jax.jit, static_argnames=())
def kernel(input_ids, word_embeddings):
    mesh = plsc.VectorSubcoreMesh(core_axis_name="c", subcore_axis_name="s")
    run = pl.kernel(
        _emb_body,
        out_type=jax.ShapeDtypeStruct((4, 4096, _D), jnp.float32),
        mesh=mesh,
        scratch_types=(
            [pltpu.VMEM((_BPW,), jnp.int32)]
            + [pltpu.VMEM((_C, _D), jnp.float32)] * _NBUF
            + [pltpu.SemaphoreType.DMA] * (2 * _NBUF)
        ),
    )
    return run(input_ids, word_embeddings)
